# Initial kernel scaffold; baseline (speedup 1.0000x reference)
import jax
import jax.numpy as jnp
from jax.experimental import pallas as pl


def kernel(x, sh, weight, CG_vals, M1, M2, seg1_ids, l_ind, seg2_ids):
    raise NotImplementedError("write your pallas kernel here")


# speedup vs baseline: 6.0476x; 6.0476x over previous
import jax
import jax.numpy as jnp
from jax.experimental import pallas as pl

L_MAX = 2
NO = (L_MAX + 1) ** 2
F = NO * 16
TN = 512


def _so3_body(x_ref, sh_ref, k_ref, out_ref):
    x = x_ref[...]
    sh = sh_ref[...]
    acc = jnp.zeros((x.shape[0], F), dtype=jnp.float32)
    for me in range(NO):
        xs = x * sh[:, me:me + 1]
        acc = acc + jnp.dot(xs, k_ref[me], preferred_element_type=jnp.float32)
    out_ref[...] = acc


def kernel(x, sh, weight, CG_vals, M1, M2, seg1_ids, l_ind, seg2_ids):
    n = x.shape[0]
    w_e = weight[0][l_ind[seg1_ids]]
    mo_e = seg2_ids[seg1_ids]
    k = jnp.zeros((NO, NO, 16, NO, 16), dtype=jnp.float32)
    k = k.at[M2, M1, :, mo_e, :].add(CG_vals[:, None, None] * w_e)
    k = k.reshape(NO, F, F)

    x_flat = x.reshape(n, F)
    out = pl.pallas_call(
        _so3_body,
        grid=(n // TN,),
        in_specs=[
            pl.BlockSpec((TN, F), lambda i: (i, 0)),
            pl.BlockSpec((TN, NO), lambda i: (i, 0)),
            pl.BlockSpec((NO, F, F), lambda i: (0, 0, 0)),
        ],
        out_specs=pl.BlockSpec((TN, F), lambda i: (i, 0)),
        out_shape=jax.ShapeDtypeStruct((n, F), jnp.float32),
    )(x_flat, sh, k)
    return out.reshape(n, NO, 16)


# speedup vs baseline: 6.3020x; 1.0421x over previous
import jax
import jax.numpy as jnp
from jax.experimental import pallas as pl

L_MAX = 2
NO = (L_MAX + 1) ** 2
F = NO * 16
TN = 512


def _so3_body(x_ref, sh_ref, k_ref, out_ref):
    x = x_ref[...]
    sh = sh_ref[...]
    acc = jnp.zeros((x.shape[0], F), dtype=jnp.float32)
    for me in range(NO):
        xs = (x * sh[:, me:me + 1]).astype(jnp.bfloat16)
        acc = acc + jnp.dot(xs, k_ref[me], preferred_element_type=jnp.float32)
    out_ref[...] = acc


def kernel(x, sh, weight, CG_vals, M1, M2, seg1_ids, l_ind, seg2_ids):
    n = x.shape[0]
    w_e = weight[0][l_ind[seg1_ids]]
    mo_e = seg2_ids[seg1_ids]
    k = jnp.zeros((NO, NO, 16, NO, 16), dtype=jnp.float32)
    k = k.at[M2, M1, :, mo_e, :].add(CG_vals[:, None, None] * w_e)
    k = k.reshape(NO, F, F).astype(jnp.bfloat16)

    x_flat = x.reshape(n, F)
    out = pl.pallas_call(
        _so3_body,
        grid=(n // TN,),
        in_specs=[
            pl.BlockSpec((TN, F), lambda i: (i, 0)),
            pl.BlockSpec((TN, NO), lambda i: (i, 0)),
            pl.BlockSpec((NO, F, F), lambda i: (0, 0, 0)),
        ],
        out_specs=pl.BlockSpec((TN, F), lambda i: (i, 0)),
        out_shape=jax.ShapeDtypeStruct((n, F), jnp.float32),
    )(x_flat, sh, k)
    return out.reshape(n, NO, 16)
